# baseline (speedup 1.0000x reference)
import functools

import jax
import jax.numpy as jnp
from jax import lax
from jax.experimental import pallas as pl
from jax.experimental.pallas import tpu as pltpu
from jax.experimental.pallas import tpu_sc as plsc

_B, _N, _D = 4, 4096, 1024
_BN = _B * _N
_NC, _NS = 2, 16
_HALF = _BN // _NC
_RPT = _HALF // _NS
_CH = 32
_NCHUNK = _RPT // _CH
_L = 16

_BIG = 2**31 - 1


def _pad_dup(v, total):
    n = v.shape[0]
    if n == total:
        return v
    return jnp.concatenate([v, jnp.broadcast_to(v[:1], (total - n,))])


def _round_up(n, m):
    return ((n + m - 1) // m) * m


def _make_sc_kernel(ngm, ngr):
    mesh = plsc.VectorSubcoreMesh(core_axis_name="c", subcore_axis_name="s")

    @functools.partial(
        pl.kernel,
        out_type=jax.ShapeDtypeStruct((_BN, _D), jnp.float32),
        mesh=mesh,
        scratch_types=[
            pltpu.VMEM((_CH, _D), jnp.float32),
            pltpu.VMEM((_CH, _D), jnp.float32),
            pltpu.VMEM((ngm, _L), jnp.int32),
            pltpu.VMEM((ngr, 1), jnp.int32),
            pltpu.VMEM((ngr, 1), jnp.int32),
            pltpu.VMEM((ngr,), jnp.int32),
            pltpu.VMEM((_L, _D), jnp.float32),
        ]
        + [pltpu.VMEM((1, _D), jnp.float32) for _ in range(ngr)]
        + [pltpu.SemaphoreType.DMA] * 12,
    )
    def sc_kernel(x_hbm, tok_hbm, fm_hbm, fr_hbm, fr1_hbm, rs_hbm, y_hbm,
                  buf0, buf1, midx, ridx, rsrc, rtv, tok_v, *rest):
        rrows = rest[:ngr]
        (sa, sb, sc2, sd, se, rs0, rs1, ws0, ws1, st0, st1, sr) = rest[ngr:]
        c = lax.axis_index("c")
        s = lax.axis_index("s")
        base_half = c * _HALF
        row0 = base_half + s * _RPT

        st_midx = pltpu.async_copy(fm_hbm.at[s], midx, sa)
        st_ridx = pltpu.async_copy(fr_hbm.at[s], ridx, sb)
        st_rsrc = pltpu.async_copy(rs_hbm.at[s], rsrc, sc2)
        st_rtv = pltpu.async_copy(fr1_hbm.at[s], rtv, sb)
        st_tok = pltpu.async_copy(tok_hbm, tok_v, sd)
        st_rsrc.wait()
        for j in range(ngr):
            pltpu.async_copy(x_hbm.at[rsrc.at[j]], rrows[j], se)

        bufs = (buf0, buf1)
        rsems = (rs0, rs1)
        wsems = (ws0, ws1)

        def mk_rd(i):
            return pltpu.make_async_copy(
                x_hbm.at[pl.ds(row0 + i * _CH, _CH)], bufs[i % 2], rsems[i % 2])

        def mk_wr(i):
            return pltpu.make_async_copy(
                bufs[i % 2], y_hbm.at[pl.ds(row0 + i * _CH, _CH)], wsems[i % 2])

        mk_rd(0).start()
        for i in range(_NCHUNK):
            mk_rd(i).wait()
            mk_wr(i).start()
            if i + 1 < _NCHUNK:
                if i >= 1:
                    mk_wr(i - 1).wait()
                mk_rd(i + 1).start()
        mk_wr(_NCHUNK - 2).wait()
        mk_wr(_NCHUNK - 1).wait()

        st_midx.wait()
        st_tok.wait()
        st_ridx.wait()
        st_rtv.wait()
        lim = base_half + _HALF
        fb = jnp.int32(_BIG)
        for j in range(ngm):
            v = midx[j, :]
            m = (v >= base_half) & (v < lim)
            sel = jnp.where(m, v, jnp.int32(_BIG))
            for k in range(_L):
                fb = jnp.minimum(fb, sel[k])
        fbv = jnp.full((_L,), fb, jnp.int32)
        for j in range(ngm):
            v = midx[j, :]
            m = (v >= base_half) & (v < lim)
            midx[j, :] = jnp.where(m, v, fbv)
        have_tok = fb < _BIG

        for j in range(ngr):
            pltpu.make_async_copy(x_hbm.at[rsrc.at[j]], rrows[j], se).wait()
        plsc.subcore_barrier()

        @pl.when(have_tok)
        def _():
            for j in range(ngm):
                pltpu.async_copy(tok_v, y_hbm.at[midx.at[j]],
                                 st0 if j % 2 == 0 else st1)

        rv = rtv[...]
        for j in range(ngr):
            t = rv[j]
            mine = (t >= base_half) & (t < lim)

            @pl.when(mine)
            def _():
                pltpu.async_copy(rrows[j], y_hbm.at[ridx.at[j]], sr)

        @pl.when(have_tok)
        def _():
            for j in range(ngm):
                pltpu.make_async_copy(tok_v, y_hbm.at[midx.at[j]],
                                      st0 if j % 2 == 0 else st1).wait()

        for j in range(ngr):
            t = rv[j]
            mine = (t >= base_half) & (t < lim)

            @pl.when(mine)
            def _():
                pltpu.make_async_copy(rrows[j], y_hbm.at[ridx.at[j]], sr).wait()

    return sc_kernel


def kernel(x, mask_token, mask, idx_b_m, idx_n_m, idx_b_r, idx_n_r, rand_b, rand_n):
    xf = x.reshape(_BN, _D)

    num_mask = idx_b_m.shape[0]
    num_rand = idx_b_r.shape[0]
    m_pad = _round_up(max(num_mask, 1), _L * _NS)
    r_pad = _round_up(max(num_rand, 1), _NS)
    ngm = m_pad // (_NS * _L)
    ngr = r_pad // _NS

    flat_m = _pad_dup(idx_b_m * _N + idx_n_m, m_pad).reshape(_NS, ngm, _L)
    flat_r = _pad_dup(idx_b_r * _N + idx_n_r, r_pad).reshape(_NS, ngr, 1)
    rand_src = _pad_dup(rand_b * _N + rand_n, r_pad).reshape(_NS, ngr, 1)
    tok_chunk = jnp.broadcast_to(mask_token.reshape(1, _D), (_L, _D))

    y = _make_sc_kernel(ngm, ngr)(xf, tok_chunk, flat_m, flat_r,
                                  flat_r.reshape(_NS, ngr), rand_src)
    return y.reshape(_B, _N, _D), mask


# speedup vs baseline: 1.1952x; 1.1952x over previous
import functools

import jax
import jax.numpy as jnp
from jax import lax
from jax.experimental import pallas as pl
from jax.experimental.pallas import tpu as pltpu
from jax.experimental.pallas import tpu_sc as plsc

_B, _N, _D = 4, 4096, 1024
_BN = _B * _N
_NC, _NS = 2, 16
_NW = _NC * _NS

_COPY_ROWS = 2048


def _copy_body(x_ref, o_ref):
    o_ref[...] = x_ref[...]


def _tc_copy(xf):
    return pl.pallas_call(
        _copy_body,
        grid=(_BN // _COPY_ROWS,),
        in_specs=[pl.BlockSpec((_COPY_ROWS, _D), lambda i: (i, 0))],
        out_specs=pl.BlockSpec((_COPY_ROWS, _D), lambda i: (i, 0)),
        out_shape=jax.ShapeDtypeStruct((_BN, _D), jnp.float32),
    )(xf)


def _pad_dup(v, total):
    n = v.shape[0]
    if n == total:
        return v
    return jnp.concatenate([v, jnp.broadcast_to(v[:1], (total - n,))])


_TG = 16


def _make_sc_scatter(cm, cr):
    mesh = plsc.VectorSubcoreMesh(core_axis_name="c", subcore_axis_name="s")
    ng = cm // _TG

    @functools.partial(
        pl.kernel,
        out_type=(),
        mesh=mesh,
        scratch_types=[
            pltpu.VMEM((ng, _TG), jnp.int32),
            pltpu.VMEM((cr,), jnp.int32),
            pltpu.VMEM((cr,), jnp.int32),
            pltpu.VMEM((_TG, _D), jnp.float32),
            pltpu.VMEM((cr, _D), jnp.float32),
            pltpu.SemaphoreType.DMA,
            pltpu.SemaphoreType.DMA,
            pltpu.SemaphoreType.DMA,
            pltpu.SemaphoreType.DMA,
        ],
    )
    def sc_scatter(y_ref, x_hbm, tok_hbm, fm_hbm, fr_hbm, rs_hbm,
                   midx_v, ridx_v, rsrc_v, tok_v, rrow_v, s0, s1, s2, s3):
        wid = lax.axis_index("s") * _NC + lax.axis_index("c")
        ld0 = pltpu.async_copy(fm_hbm.at[wid], midx_v, s0)
        ld1 = pltpu.async_copy(fr_hbm.at[pl.ds(wid * cr, cr)], ridx_v, s1)
        ld2 = pltpu.async_copy(rs_hbm.at[pl.ds(wid * cr, cr)], rsrc_v, s2)
        ld3 = pltpu.async_copy(tok_hbm, tok_v, s3)
        ld2.wait()
        g = pltpu.async_copy(x_hbm.at[rsrc_v], rrow_v, s2)
        ld0.wait()
        ld3.wait()
        sems = (s0, s3)
        toks = []
        for j in range(ng):
            toks.append(pltpu.async_copy(tok_v, y_ref.at[midx_v.at[j]],
                                         sems[j % 2]))
        ld1.wait()
        g.wait()
        cp2 = pltpu.async_copy(rrow_v, y_ref.at[ridx_v], s1)
        for c in toks:
            c.wait()
        cp2.wait()

    return sc_scatter


def _round_up(n, m):
    return ((n + m - 1) // m) * m


def kernel(x, mask_token, mask, idx_b_m, idx_n_m, idx_b_r, idx_n_r, rand_b, rand_n):
    xf = x.reshape(_BN, _D)

    num_mask = idx_b_m.shape[0]
    num_rand = idx_b_r.shape[0]
    m_pad = _round_up(max(num_mask, 1), 8 * _NW)
    r_pad = _round_up(max(num_rand, 1), 8 * _NW)
    cm = m_pad // _NW
    cr = r_pad // _NW

    flat_m = _pad_dup(idx_b_m * _N + idx_n_m, m_pad).reshape(_NW, cm // _TG, _TG)
    flat_r = _pad_dup(idx_b_r * _N + idx_n_r, r_pad)
    rand_src = _pad_dup(rand_b * _N + rand_n, r_pad)
    tok_chunk = jnp.broadcast_to(mask_token.reshape(1, _D), (_TG, _D))

    y = _tc_copy(xf)
    y_ref = jax.new_ref(y)
    _make_sc_scatter(cm, cr)(y_ref, xf, tok_chunk, flat_m, flat_r, rand_src)
    out = jax.freeze(y_ref)
    return out.reshape(_B, _N, _D), mask


# speedup vs baseline: 1.2165x; 1.0178x over previous
import functools

import jax
import jax.numpy as jnp
from jax import lax
from jax.experimental import pallas as pl
from jax.experimental.pallas import tpu as pltpu
from jax.experimental.pallas import tpu_sc as plsc

_B, _N, _D = 4, 4096, 1024
_BN = _B * _N
_NC, _NS = 2, 16
_NW = _NC * _NS

_COPY_ROWS = 2048


def _copy_body(x_ref, o_ref):
    o_ref[...] = x_ref[...]


def _tc_copy(xf):
    return pl.pallas_call(
        _copy_body,
        grid=(_BN // _COPY_ROWS,),
        in_specs=[pl.BlockSpec((_COPY_ROWS, _D), lambda i: (i, 0))],
        out_specs=pl.BlockSpec((_COPY_ROWS, _D), lambda i: (i, 0)),
        out_shape=jax.ShapeDtypeStruct((_BN, _D), jnp.float32),
    )(xf)


def _pad_dup(v, total):
    n = v.shape[0]
    if n == total:
        return v
    return jnp.concatenate([v, jnp.broadcast_to(v[:1], (total - n,))])


_TG = 8


def _make_sc_scatter(cm, cr):
    mesh = plsc.VectorSubcoreMesh(core_axis_name="c", subcore_axis_name="s")
    ng = cm // _TG

    @functools.partial(
        pl.kernel,
        out_type=(),
        mesh=mesh,
        scratch_types=[
            pltpu.VMEM((ng, _TG), jnp.int32),
            pltpu.VMEM((cr,), jnp.int32),
            pltpu.VMEM((cr,), jnp.int32),
            pltpu.VMEM((_TG, _D), jnp.float32),
            pltpu.VMEM((cr, _D), jnp.float32),
            pltpu.SemaphoreType.DMA,
            pltpu.SemaphoreType.DMA,
            pltpu.SemaphoreType.DMA,
            pltpu.SemaphoreType.DMA,
        ],
    )
    def sc_scatter(y_ref, x_hbm, tok_hbm, fm_hbm, fr_hbm, rs_hbm,
                   midx_v, ridx_v, rsrc_v, tok_v, rrow_v, s0, s1, s2, s3):
        wid = lax.axis_index("s") * _NC + lax.axis_index("c")
        ld0 = pltpu.async_copy(fm_hbm.at[wid], midx_v, s0)
        ld1 = pltpu.async_copy(fr_hbm.at[pl.ds(wid * cr, cr)], ridx_v, s1)
        ld2 = pltpu.async_copy(rs_hbm.at[pl.ds(wid * cr, cr)], rsrc_v, s2)
        ld3 = pltpu.async_copy(tok_hbm, tok_v, s3)
        ld2.wait()
        g = pltpu.async_copy(x_hbm.at[rsrc_v], rrow_v, s2)
        ld0.wait()
        ld3.wait()
        toks = []
        for j in range(ng):
            toks.append(pltpu.async_copy(tok_v, y_ref.at[midx_v.at[j]],
                                         s0 if j % 2 == 0 else s3))
        ld1.wait()
        g.wait()
        cp2 = pltpu.async_copy(rrow_v, y_ref.at[ridx_v], s1)
        for c in toks:
            c.wait()
        cp2.wait()

    return sc_scatter


def _round_up(n, m):
    return ((n + m - 1) // m) * m


def kernel(x, mask_token, mask, idx_b_m, idx_n_m, idx_b_r, idx_n_r, rand_b, rand_n):
    xf = x.reshape(_BN, _D)

    num_mask = idx_b_m.shape[0]
    num_rand = idx_b_r.shape[0]
    m_pad = _round_up(max(num_mask, 1), 8 * _NW)
    r_pad = _round_up(max(num_rand, 1), 8 * _NW)
    cm = m_pad // _NW
    cr = r_pad // _NW

    flat_m = _pad_dup(idx_b_m * _N + idx_n_m, m_pad).reshape(_NW, cm // _TG, _TG)
    flat_r = _pad_dup(idx_b_r * _N + idx_n_r, r_pad)
    rand_src = _pad_dup(rand_b * _N + rand_n, r_pad)
    tok_chunk = jnp.broadcast_to(mask_token.reshape(1, _D), (_TG, _D))

    y = _tc_copy(xf)
    y_ref = jax.new_ref(y)
    _make_sc_scatter(cm, cr)(y_ref, xf, tok_chunk, flat_m, flat_r, rand_src)
    out = jax.freeze(y_ref)
    return out.reshape(_B, _N, _D), mask
